# baseline (speedup 1.0000x reference)
import jax
import jax.numpy as jnp
from jax.experimental import pallas as pl
from jax.experimental.pallas import tpu as pltpu

_K = 64
_SOLVE_ITERS = 5
_ROWS_PER_BLOCK = 32


def _token_selector_block(x_ref, w_ref, n_ref):
    x = x_ref[...]
    r, d = x.shape
    m0 = jnp.max(x, axis=-1, keepdims=True)

    ones = jnp.ones((d, 1), dtype=jnp.float32)

    def rowsum(a):
        return jax.lax.dot_general(
            a, ones, (((1,), (0,)), ((), ())),
            precision=jax.lax.Precision.HIGHEST,
            preferred_element_type=jnp.float32)

    def solve_body(_, carry):
        tau, _ = carry
        mx = jnp.where(x > tau, x, 0.0)
        act = jnp.where(x > tau, 1.0, 0.0)
        cnt = rowsum(act)
        s1 = rowsum(mx)
        s2 = rowsum(mx * mx)
        mean = s1 / cnt
        delta = mean * mean - (s2 - 4.0) / cnt
        return mean - jnp.sqrt(jnp.maximum(delta, 0.0)), cnt

    cnt0 = jnp.zeros((r, 1), dtype=jnp.float32)
    tau, support = jax.lax.fori_loop(
        0, _SOLVE_ITERS, solve_body, (m0 - 1.0, cnt0))
    y = jnp.maximum(x - tau, 0.0)
    y = y * y

    def finalize(w):
        s = rowsum(w)
        wn = w / (s + 4e-8)
        w_ref[...] = wn
        n_ref[...] = rowsum(jnp.where(wn > 1e-6, 1.0, 0.0)).astype(jnp.int32)

    need_topk = jnp.any(support > jnp.float32(_K))

    @pl.when(jnp.logical_not(need_topk))
    def _():
        finalize(y)

    @pl.when(need_topk)
    def _():
        X = x - m0
        xi = jax.lax.bitcast_convert_type(X, jnp.int32)
        key = jnp.where(xi < 0, xi ^ jnp.int32(0x7FFFFFFF), xi)

        def val_body(_, lo_hi):
            lo, hi = lo_hi
            mid = hi - ((hi - lo) >> 1)
            cnt = jnp.sum((key >= mid).astype(jnp.float32), axis=-1,
                          keepdims=True)
            take = cnt >= _K
            return jnp.where(take, mid, lo), jnp.where(take, hi, mid - 1)

        lo0 = jnp.full((r, 1), jnp.iinfo(jnp.int32).min + 1, dtype=jnp.int32)
        hi0 = jnp.zeros((r, 1), dtype=jnp.int32)
        t64, _ = jax.lax.fori_loop(0, 31, val_body, (lo0, hi0))

        c_gt = jnp.sum((key > t64).astype(jnp.float32), axis=-1,
                       keepdims=True)
        keep_eq = jnp.float32(_K) - c_gt
        is_eq = key == t64
        is_eqf = is_eq.astype(jnp.float32)
        idx = jax.lax.broadcasted_iota(jnp.int32, (r, d), 1)

        def idx_body(_, lo_hi):
            ilo, ihi = lo_hi
            mid = ilo + ((ihi - ilo) >> 1)
            cnt = jnp.sum(jnp.where(idx <= mid, is_eqf, 0.0), axis=-1,
                          keepdims=True)
            ok = cnt >= keep_eq
            return jnp.where(ok, ilo, mid + 1), jnp.where(ok, mid, ihi)

        ilo0 = jnp.zeros((r, 1), dtype=jnp.int32)
        ihi0 = jnp.full((r, 1), d - 1, dtype=jnp.int32)
        _, idx_thr = jax.lax.fori_loop(0, 15, idx_body, (ilo0, ihi0))

        keep = (key > t64) | (is_eq & (idx <= idx_thr))
        finalize(jnp.where(keep, y, 0.0))


def kernel(logits):
    b, d = logits.shape
    r = _ROWS_PER_BLOCK
    w, n = pl.pallas_call(
        _token_selector_block,
        grid=(b // r,),
        in_specs=[pl.BlockSpec((r, d), lambda i: (i, 0))],
        out_specs=[
            pl.BlockSpec((r, d), lambda i: (i, 0)),
            pl.BlockSpec((r, 1), lambda i: (i, 0)),
        ],
        out_shape=[
            jax.ShapeDtypeStruct((b, d), jnp.float32),
            jax.ShapeDtypeStruct((b, 1), jnp.int32),
        ],
        compiler_params=pltpu.CompilerParams(
            dimension_semantics=("parallel",)),
    )(logits)
    return w, n.reshape(b)


# speedup vs baseline: 11.8172x; 11.8172x over previous
import jax
import jax.numpy as jnp
from jax.experimental import pallas as pl
from jax.experimental.pallas import tpu as pltpu

_K = 64
_SOLVE_ITERS = 4
_ROWS_PER_BLOCK = 32


def _token_selector_block(x_ref, w_ref, n_ref):
    x = x_ref[...]
    r, d = x.shape
    m0 = jnp.max(x, axis=-1, keepdims=True)

    def solve_body(_, carry):
        tau, _ = carry
        mx = jnp.where(x > tau, x, 0.0)
        act = jnp.where(x > tau, 1.0, 0.0)
        cnt = jnp.sum(act, axis=-1, keepdims=True)
        s1 = jnp.sum(mx, axis=-1, keepdims=True)
        s2 = jnp.sum(mx * mx, axis=-1, keepdims=True)
        mean = s1 / cnt
        delta = mean * mean - (s2 - 4.0) / cnt
        return mean - jnp.sqrt(jnp.maximum(delta, 0.0)), cnt

    cnt0 = jnp.zeros((r, 1), dtype=jnp.float32)
    tau, support = jax.lax.fori_loop(
        0, _SOLVE_ITERS, solve_body, (m0 - 1.0, cnt0))
    y = jnp.maximum(x - tau, 0.0)
    y = y * y

    def finalize(w):
        s = jnp.sum(w, axis=-1, keepdims=True)
        wn = w / (s + 4e-8)
        w_ref[...] = wn
        n_ref[...] = jnp.sum((wn > 1e-6).astype(jnp.int32), axis=-1,
                             keepdims=True)

    need_topk = jnp.any(support > jnp.float32(_K))

    @pl.when(jnp.logical_not(need_topk))
    def _():
        finalize(y)

    @pl.when(need_topk)
    def _():
        X = x - m0
        xi = jax.lax.bitcast_convert_type(X, jnp.int32)
        key = jnp.where(xi < 0, xi ^ jnp.int32(0x7FFFFFFF), xi)

        def val_body(_, lo_hi):
            lo, hi = lo_hi
            mid = hi - ((hi - lo) >> 1)
            cnt = jnp.sum((key >= mid).astype(jnp.float32), axis=-1,
                          keepdims=True)
            take = cnt >= _K
            return jnp.where(take, mid, lo), jnp.where(take, hi, mid - 1)

        lo0 = jnp.full((r, 1), jnp.iinfo(jnp.int32).min + 1, dtype=jnp.int32)
        hi0 = jnp.zeros((r, 1), dtype=jnp.int32)
        t64, _ = jax.lax.fori_loop(0, 31, val_body, (lo0, hi0))

        c_gt = jnp.sum((key > t64).astype(jnp.float32), axis=-1,
                       keepdims=True)
        keep_eq = jnp.float32(_K) - c_gt
        is_eq = key == t64
        is_eqf = is_eq.astype(jnp.float32)
        idx = jax.lax.broadcasted_iota(jnp.int32, (r, d), 1)

        def idx_body(_, lo_hi):
            ilo, ihi = lo_hi
            mid = ilo + ((ihi - ilo) >> 1)
            cnt = jnp.sum(jnp.where(idx <= mid, is_eqf, 0.0), axis=-1,
                          keepdims=True)
            ok = cnt >= keep_eq
            return jnp.where(ok, ilo, mid + 1), jnp.where(ok, mid, ihi)

        ilo0 = jnp.zeros((r, 1), dtype=jnp.int32)
        ihi0 = jnp.full((r, 1), d - 1, dtype=jnp.int32)
        _, idx_thr = jax.lax.fori_loop(0, 15, idx_body, (ilo0, ihi0))

        keep = (key > t64) | (is_eq & (idx <= idx_thr))
        finalize(jnp.where(keep, y, 0.0))


def kernel(logits):
    b, d = logits.shape
    r = _ROWS_PER_BLOCK
    w, n = pl.pallas_call(
        _token_selector_block,
        grid=(b // r,),
        in_specs=[pl.BlockSpec((r, d), lambda i: (i, 0))],
        out_specs=[
            pl.BlockSpec((r, d), lambda i: (i, 0)),
            pl.BlockSpec((r, 1), lambda i: (i, 0)),
        ],
        out_shape=[
            jax.ShapeDtypeStruct((b, d), jnp.float32),
            jax.ShapeDtypeStruct((b, 1), jnp.int32),
        ],
        compiler_params=pltpu.CompilerParams(
            dimension_semantics=("parallel",),
),
    )(logits)
    return w, n.reshape(b)
